# baseline (speedup 1.0000x reference)
import functools

import jax
import jax.numpy as jnp
from jax import lax
from jax.experimental import pallas as pl
from jax.experimental.pallas import tpu as pltpu
from jax.experimental.pallas import tpu_sc as plsc

D = 64
B_TOTAL = 69632 * 5
NC, NS = 2, 16
NW = NC * NS
CHUNK = 128
CH_PER_G = 5
GROUPS = 17
B_PER_W = CHUNK * CH_PER_G * GROUPS
assert B_PER_W * NW == B_TOTAL

_mesh = plsc.VectorSubcoreMesh(core_axis_name="c", subcore_axis_name="s")


@functools.partial(
    pl.kernel,
    mesh=_mesh,
    out_type=jax.ShapeDtypeStruct((B_TOTAL, D), jnp.float32),
    scratch_types=[
        pltpu.VMEM((GROUPS * CH_PER_G, CHUNK), jnp.int32),
        pltpu.VMEM((CH_PER_G * CHUNK, D), jnp.float32),
        pltpu.SemaphoreType.DMA,
    ],
    compiler_params=pltpu.CompilerParams(use_tc_tiling_on_sc=False),
)
def _gather_kernel(table_hbm, idx_hbm, out_hbm, idx_v, rows_v, sem):
    wid = lax.axis_index("s") * NC + lax.axis_index("c")
    pltpu.sync_copy(idx_hbm.at[wid], idx_v)
    base = wid * B_PER_W

    def body(g, carry):
        copies = []
        for j in range(CH_PER_G):
            c = pltpu.async_copy(
                table_hbm.at[idx_v.at[g * CH_PER_G + j]],
                rows_v.at[pl.ds(j * CHUNK, CHUNK)],
                sem,
            )
            copies.append(c)
        for c in copies:
            c.wait()
        pltpu.sync_copy(
            rows_v, out_hbm.at[pl.ds(base + g * (CH_PER_G * CHUNK), CH_PER_G * CHUNK)]
        )
        return carry

    lax.fori_loop(0, GROUPS, body, 0)


def kernel(values, table):
    idx = values.reshape(NW, GROUPS * CH_PER_G, CHUNK)
    return _gather_kernel(table, idx)


# speedup vs baseline: 1.0015x; 1.0015x over previous
import functools

import jax
import jax.numpy as jnp
from jax import lax
from jax.experimental import pallas as pl
from jax.experimental.pallas import tpu as pltpu
from jax.experimental.pallas import tpu_sc as plsc

D = 64
B_TOTAL = 69632 * 5
NC, NS = 2, 16
NW = NC * NS
CHUNK = 640
CH_PER_G = 1
GROUPS = 17
B_PER_W = CHUNK * CH_PER_G * GROUPS
assert B_PER_W * NW == B_TOTAL

_mesh = plsc.VectorSubcoreMesh(core_axis_name="c", subcore_axis_name="s")


@functools.partial(
    pl.kernel,
    mesh=_mesh,
    out_type=jax.ShapeDtypeStruct((B_TOTAL, D), jnp.float32),
    scratch_types=[
        pltpu.VMEM((GROUPS * CH_PER_G, CHUNK), jnp.int32),
        pltpu.VMEM((CH_PER_G * CHUNK, D), jnp.float32),
        pltpu.SemaphoreType.DMA,
    ],
    compiler_params=pltpu.CompilerParams(use_tc_tiling_on_sc=False),
)
def _gather_kernel(table_hbm, idx_hbm, out_hbm, idx_v, rows_v, sem):
    wid = lax.axis_index("s") * NC + lax.axis_index("c")
    pltpu.sync_copy(idx_hbm.at[wid], idx_v)
    base = wid * B_PER_W

    def body(g, carry):
        copies = []
        for j in range(CH_PER_G):
            c = pltpu.async_copy(
                table_hbm.at[idx_v.at[g * CH_PER_G + j]],
                rows_v.at[pl.ds(j * CHUNK, CHUNK)],
                sem,
            )
            copies.append(c)
        for c in copies:
            c.wait()
        pltpu.sync_copy(
            rows_v, out_hbm.at[pl.ds(base + g * (CH_PER_G * CHUNK), CH_PER_G * CHUNK)]
        )
        return carry

    lax.fori_loop(0, GROUPS, body, 0)


def kernel(values, table):
    idx = values.reshape(NW, GROUPS * CH_PER_G, CHUNK)
    return _gather_kernel(table, idx)


# speedup vs baseline: 1.0113x; 1.0099x over previous
import functools

import jax
import jax.numpy as jnp
from jax import lax
from jax.experimental import pallas as pl
from jax.experimental.pallas import tpu as pltpu
from jax.experimental.pallas import tpu_sc as plsc

D = 64
B_TOTAL = 69632 * 5
NC, NS = 2, 16
NW = NC * NS
CHUNK = 640
GROUPS = 17
B_PER_W = CHUNK * GROUPS
assert B_PER_W * NW == B_TOTAL

_mesh = plsc.VectorSubcoreMesh(core_axis_name="c", subcore_axis_name="s")


@functools.partial(
    pl.kernel,
    mesh=_mesh,
    out_type=jax.ShapeDtypeStruct((B_TOTAL, D), jnp.float32),
    scratch_types=[
        pltpu.VMEM((B_PER_W,), jnp.int32),
        pltpu.VMEM((2, CHUNK, D), jnp.float32),
        pltpu.SemaphoreType.DMA,
        pltpu.SemaphoreType.DMA,
        pltpu.SemaphoreType.DMA,
        pltpu.SemaphoreType.DMA,
    ],
    compiler_params=pltpu.CompilerParams(use_tc_tiling_on_sc=False),
)
def _gather_kernel(table_hbm, idx_hbm, out_hbm, idx_v, rows_v, sg0, sg1, ss0, ss1):
    wid = lax.axis_index("s") * NC + lax.axis_index("c")
    pltpu.sync_copy(idx_hbm.at[wid], idx_v)
    base = wid * B_PER_W
    sem_g = (sg0, sg1)
    sem_s = (ss0, ss1)

    g_copies = [None] * GROUPS
    s_copies = [None] * GROUPS
    for g in range(GROUPS):
        b = g % 2
        if g >= 2:
            s_copies[g - 2].wait()
        g_copies[g] = pltpu.async_copy(
            table_hbm.at[idx_v.at[pl.ds(g * CHUNK, CHUNK)]], rows_v.at[b], sem_g[b]
        )
        if g >= 1:
            pb = (g - 1) % 2
            g_copies[g - 1].wait()
            s_copies[g - 1] = pltpu.async_copy(
                rows_v.at[pb],
                out_hbm.at[pl.ds(base + (g - 1) * CHUNK, CHUNK)],
                sem_s[pb],
            )
    lb = (GROUPS - 1) % 2
    g_copies[GROUPS - 1].wait()
    s_copies[GROUPS - 1] = pltpu.async_copy(
        rows_v.at[lb],
        out_hbm.at[pl.ds(base + (GROUPS - 1) * CHUNK, CHUNK)],
        sem_s[lb],
    )
    s_copies[GROUPS - 2].wait()
    s_copies[GROUPS - 1].wait()


def kernel(values, table):
    idx = values.reshape(NW, B_PER_W)
    return _gather_kernel(table, idx)


# speedup vs baseline: 1.0115x; 1.0001x over previous
import functools

import jax
import jax.numpy as jnp
from jax import lax
from jax.experimental import pallas as pl
from jax.experimental.pallas import tpu as pltpu
from jax.experimental.pallas import tpu_sc as plsc

D = 64
B_TOTAL = 69632 * 5
NC, NS = 2, 16
NW = NC * NS
CHUNK = 640
GROUPS = 17
B_PER_W = CHUNK * GROUPS
assert B_PER_W * NW == B_TOTAL

_mesh = plsc.VectorSubcoreMesh(core_axis_name="c", subcore_axis_name="s")


@functools.partial(
    pl.kernel,
    mesh=_mesh,
    out_type=jax.ShapeDtypeStruct((B_TOTAL, D), jnp.float32),
    scratch_types=[
        pltpu.VMEM((B_PER_W,), jnp.int32),
        pltpu.VMEM((2, CHUNK, D), jnp.float32),
        pltpu.SemaphoreType.DMA,
        pltpu.SemaphoreType.DMA,
        pltpu.SemaphoreType.DMA,
        pltpu.SemaphoreType.DMA,
    ],
    compiler_params=pltpu.CompilerParams(use_tc_tiling_on_sc=False),
)
def _gather_kernel(table_hbm, idx_hbm, out_hbm, idx_v, rows_v, sg0, sg1, ss0, ss1):
    wid = lax.axis_index("s") * NC + lax.axis_index("c")
    pltpu.sync_copy(idx_hbm.at[pl.ds(wid * B_PER_W, B_PER_W)], idx_v)
    base = wid * B_PER_W
    sem_g = (sg0, sg1)
    sem_s = (ss0, ss1)

    g_copies = [None] * GROUPS
    s_copies = [None] * GROUPS
    for g in range(GROUPS):
        b = g % 2
        if g >= 2:
            s_copies[g - 2].wait()
        g_copies[g] = pltpu.async_copy(
            table_hbm.at[idx_v.at[pl.ds(g * CHUNK, CHUNK)]], rows_v.at[b], sem_g[b]
        )
        if g >= 1:
            pb = (g - 1) % 2
            g_copies[g - 1].wait()
            s_copies[g - 1] = pltpu.async_copy(
                rows_v.at[pb],
                out_hbm.at[pl.ds(base + (g - 1) * CHUNK, CHUNK)],
                sem_s[pb],
            )
    lb = (GROUPS - 1) % 2
    g_copies[GROUPS - 1].wait()
    s_copies[GROUPS - 1] = pltpu.async_copy(
        rows_v.at[lb],
        out_hbm.at[pl.ds(base + (GROUPS - 1) * CHUNK, CHUNK)],
        sem_s[lb],
    )
    s_copies[GROUPS - 2].wait()
    s_copies[GROUPS - 1].wait()


def kernel(values, table):
    idx = values.reshape(-1)
    return _gather_kernel(table, idx)


# speedup vs baseline: 1.1837x; 1.1703x over previous
import functools

import jax
import jax.numpy as jnp
from jax import lax
from jax.experimental import pallas as pl
from jax.experimental.pallas import tpu as pltpu
from jax.experimental.pallas import tpu_sc as plsc

D = 64
DP = 128
B_TOTAL = 69632 * 5
NC, NS = 2, 16
NW = NC * NS
CHUNK = 320
GROUPS = 34
B_PER_W = CHUNK * GROUPS
assert B_PER_W * NW == B_TOTAL

_mesh = plsc.VectorSubcoreMesh(core_axis_name="c", subcore_axis_name="s")


@functools.partial(
    pl.kernel,
    mesh=_mesh,
    out_type=jax.ShapeDtypeStruct((B_TOTAL, DP), jnp.float32),
    scratch_types=[
        pltpu.VMEM((B_PER_W,), jnp.int32),
        pltpu.VMEM((2, CHUNK, DP), jnp.float32),
        pltpu.SemaphoreType.DMA,
        pltpu.SemaphoreType.DMA,
        pltpu.SemaphoreType.DMA,
        pltpu.SemaphoreType.DMA,
    ],
    compiler_params=pltpu.CompilerParams(use_tc_tiling_on_sc=False),
)
def _gather_kernel(table_hbm, idx_hbm, out_hbm, idx_v, rows_v, sg0, sg1, ss0, ss1):
    wid = lax.axis_index("s") * NC + lax.axis_index("c")
    pltpu.sync_copy(idx_hbm.at[pl.ds(wid * B_PER_W, B_PER_W)], idx_v)
    base = wid * B_PER_W
    sem_g = (sg0, sg1)
    sem_s = (ss0, ss1)

    g_copies = [None] * GROUPS
    s_copies = [None] * GROUPS
    for g in range(GROUPS):
        b = g % 2
        if g >= 2:
            s_copies[g - 2].wait()
        g_copies[g] = pltpu.async_copy(
            table_hbm.at[idx_v.at[pl.ds(g * CHUNK, CHUNK)]], rows_v.at[b], sem_g[b]
        )
        if g >= 1:
            pb = (g - 1) % 2
            g_copies[g - 1].wait()
            s_copies[g - 1] = pltpu.async_copy(
                rows_v.at[pb],
                out_hbm.at[pl.ds(base + (g - 1) * CHUNK, CHUNK)],
                sem_s[pb],
            )
    lb = (GROUPS - 1) % 2
    g_copies[GROUPS - 1].wait()
    s_copies[GROUPS - 1] = pltpu.async_copy(
        rows_v.at[lb],
        out_hbm.at[pl.ds(base + (GROUPS - 1) * CHUNK, CHUNK)],
        sem_s[lb],
    )
    s_copies[GROUPS - 2].wait()
    s_copies[GROUPS - 1].wait()


def kernel(values, table):
    tpad = jnp.pad(table, ((0, 7), (0, DP - D)))
    idx = values.reshape(-1)
    out_pad = _gather_kernel(tpad, idx)
    return out_pad[:, :D]
